# baseline (speedup 1.0000x reference)
import functools

import jax
import jax.numpy as jnp
from jax import lax
from jax.experimental import pallas as pl
from jax.experimental.pallas import tpu as pltpu
from jax.experimental.pallas import tpu_sc as plsc

B, L, D, K, W = 16, 512, 1024, 32, 5
LK = L - K
BIG = 1 << 30
NEG = -1e30


def _proj_body(doc_ref, we_ref, be_ref, wc_ref, bc_ref, wo_ref, bo_ref,
               emo_ref, con_ref, pred_ref):
    x = doc_ref[0]
    dn = (((1,), (1,)), ((), ()))
    emo = lax.dot_general(x, we_ref[...], dn,
                          preferred_element_type=jnp.float32) + be_ref[...]
    con = lax.dot_general(x, wc_ref[...], dn,
                          preferred_element_type=jnp.float32) + bc_ref[...]
    emo_ref[0] = emo
    con_ref[0] = con
    pred = lax.dot_general(wo_ref[...], emo, dn,
                           preferred_element_type=jnp.float32) + bo_ref[0, 0]
    pred_ref[0] = pred


def _projections(doc, we, be, wc, bc, wo, bo):
    return pl.pallas_call(
        _proj_body,
        grid=(B,),
        in_specs=[
            pl.BlockSpec((1, L, D), lambda b: (b, 0, 0)),
            pl.BlockSpec((D, D), lambda b: (0, 0)),
            pl.BlockSpec((D,), lambda b: (0,)),
            pl.BlockSpec((D, D), lambda b: (0, 0)),
            pl.BlockSpec((D,), lambda b: (0,)),
            pl.BlockSpec((1, D), lambda b: (0, 0)),
            pl.BlockSpec((1, 1), lambda b: (0, 0)),
        ],
        out_specs=[
            pl.BlockSpec((1, L, D), lambda b: (b, 0, 0)),
            pl.BlockSpec((1, L, D), lambda b: (b, 0, 0)),
            pl.BlockSpec((1, 1, L), lambda b: (b, 0, 0)),
        ],
        out_shape=[
            jax.ShapeDtypeStruct((B, L, D), jnp.float32),
            jax.ShapeDtypeStruct((B, L, D), jnp.float32),
            jax.ShapeDtypeStruct((B, 1, L), jnp.float32),
        ],
    )(doc, we, be, wc, bc, wo, bo.reshape(1, 1))


def _index_body(pred_ref, topk_ref, no_ref, rep_t_ref, ctx_t_ref,
                rep_n_ref, ctx_n_ref):
    vals = pred_ref[...]
    iota_l = lax.broadcasted_iota(jnp.int32, (B, L), 1)
    iota_k = lax.broadcasted_iota(jnp.int32, (B, K), 1)

    def topk_step(i, carry):
        vals, topk = carry
        m = jnp.max(vals, axis=1, keepdims=True)
        idx = jnp.min(jnp.where(vals == m, iota_l, BIG), axis=1,
                      keepdims=True)
        topk = jnp.where(iota_k == i, idx, topk)
        vals = jnp.where(iota_l == idx, NEG, vals)
        return vals, topk

    topk0 = jnp.zeros((B, K), jnp.int32)
    _, topk = lax.fori_loop(0, K, topk_step, (vals, topk0))
    topk_ref[...] = topk

    def sort_step(i, carry):
        tv, srt = carry
        t = jnp.min(tv, axis=1, keepdims=True)
        srt = jnp.where(iota_k == i, t, srt)
        tv = jnp.where(tv == t, BIG, tv)
        return tv, srt

    _, srt = lax.fori_loop(0, K, sort_step, (topk, topk0))

    v0 = lax.broadcasted_iota(jnp.int32, (B, LK), 1)

    def diff_step(i, v):
        t = jnp.min(jnp.where(iota_k == i, srt, BIG), axis=1, keepdims=True)
        return v + (t <= v).astype(jnp.int32)

    no_idx = lax.fori_loop(0, K, diff_step, v0)
    no_ref[...] = no_idx

    def repeat5(ids, n):
        src = lax.broadcasted_iota(jnp.int32, (n, W * n), 0)
        dst = lax.broadcasted_iota(jnp.int32, (n, W * n), 1) // W
        r = (src == dst).astype(jnp.float32)
        rep = lax.dot_general(ids.astype(jnp.float32), r,
                              (((1,), (0,)), ((), ())),
                              precision=lax.Precision.HIGHEST,
                              preferred_element_type=jnp.float32)
        return rep.astype(jnp.int32)

    def window(rep, n):
        off = lax.broadcasted_iota(jnp.int32, (B, W * n), 1) % W - 2
        return jnp.clip(rep + off, 0, L - 1)

    rep_t = repeat5(topk, K)
    rep_n = repeat5(no_idx, LK)
    rep_t_ref[...] = rep_t
    rep_n_ref[...] = rep_n
    ctx_t_ref[...] = window(rep_t, K)
    ctx_n_ref[...] = window(rep_n, LK)


def _indices(pred):
    shapes = [(B, K), (B, LK), (B, K * W), (B, K * W), (B, LK * W),
              (B, LK * W)]
    return pl.pallas_call(
        _index_body,
        out_shape=[jax.ShapeDtypeStruct(s, jnp.int32) for s in shapes],
    )(pred)


NW = 32
_CHUNK = 40

_TASKS = (
    ("emo", K * B // NW, 16),
    ("con", K * W * B // NW, 40),
    ("emo", LK * B // NW, 40),
    ("con", LK * W * B // NW, 40),
)


def _sc_gather_body(emo_ref, con_ref, i0_ref, i1_ref, i2_ref, i3_ref,
                    o0_ref, o1_ref, o2_ref, o3_ref,
                    ix0, ix1, ix2, ix3, buf_v, sem_g, sem_o0, sem_o1):
    wid = lax.axis_index("s") * 2 + lax.axis_index("c")
    tables = {"emo": emo_ref, "con": con_ref}
    idx_refs = (i0_ref, i1_ref, i2_ref, i3_ref)
    idx_bufs = (ix0, ix1, ix2, ix3)
    out_refs = (o0_ref, o1_ref, o2_ref, o3_ref)
    sem_o = (sem_o0, sem_o1)

    for t, (_, rows, chunk) in enumerate(_TASKS):
        pltpu.sync_copy(idx_refs[t].at[pl.ds(wid * rows, rows)], idx_bufs[t])

    pending = [None, None]

    def do_chunk(tab, idx_slice, out_slice, slot, chunk):
        if pending[slot] is not None:
            pending[slot].wait()
        pltpu.async_copy(tables[tab].at[idx_slice],
                         buf_v.at[slot, pl.ds(0, chunk)], sem_g).wait()
        pending[slot] = pltpu.async_copy(buf_v.at[slot, pl.ds(0, chunk)],
                                         out_slice, sem_o[slot])

    n = 0
    for t in (0, 1, 2):
        tab, rows, chunk = _TASKS[t]
        for c in range(rows // chunk):
            do_chunk(tab,
                     idx_bufs[t].at[pl.ds(c * chunk, chunk)],
                     out_refs[t].at[pl.ds(wid * rows + c * chunk, chunk)],
                     n % 2, chunk)
            n += 1
    tab, rows, chunk = _TASKS[3]
    nch = rows // chunk

    @pl.loop(0, nch, step=2)
    def _(c0):
        for b in range(2):
            c = c0 + b
            start = pl.multiple_of(c * chunk, chunk)
            obase = pl.multiple_of(wid * rows + c * chunk, chunk)
            pltpu.make_async_copy(buf_v.at[b, pl.ds(0, chunk)],
                                  out_refs[3].at[pl.ds(0, chunk)],
                                  sem_o[b]).wait()
            pltpu.async_copy(tables[tab].at[idx_bufs[3].at[pl.ds(start, chunk)]],
                             buf_v.at[b, pl.ds(0, chunk)], sem_g).wait()
            pltpu.async_copy(buf_v.at[b, pl.ds(0, chunk)],
                             out_refs[3].at[pl.ds(obase, chunk)], sem_o[b])

    for b in range(2):
        pltpu.make_async_copy(buf_v.at[b, pl.ds(0, chunk)],
                              out_refs[3].at[pl.ds(0, chunk)],
                              sem_o[b]).wait()


def _sc_gather(emo_flat, con_flat, g_cand, g_ctx_t, g_no, g_ctx_n):
    mesh = plsc.VectorSubcoreMesh(core_axis_name="c", subcore_axis_name="s")
    kern = pl.kernel(
        _sc_gather_body,
        compiler_params=pltpu.CompilerParams(use_tc_tiling_on_sc=True),
        out_type=[
            jax.ShapeDtypeStruct((B * K, D), jnp.float32),
            jax.ShapeDtypeStruct((B * K * W, D), jnp.float32),
            jax.ShapeDtypeStruct((B * LK, D), jnp.float32),
            jax.ShapeDtypeStruct((B * LK * W, D), jnp.float32),
        ],
        mesh=mesh,
        scratch_types=[
            pltpu.VMEM((K * B // NW,), jnp.int32),
            pltpu.VMEM((K * W * B // NW,), jnp.int32),
            pltpu.VMEM((LK * B // NW,), jnp.int32),
            pltpu.VMEM((LK * W * B // NW,), jnp.int32),
            pltpu.VMEM((2, _CHUNK, D), jnp.float32),
            pltpu.SemaphoreType.DMA,
            pltpu.SemaphoreType.DMA,
            pltpu.SemaphoreType.DMA,
        ],
    )
    return kern(emo_flat, con_flat, g_cand, g_ctx_t, g_no, g_ctx_n)


def kernel(doc_sents_h, W_emo, b_emo, W_con, b_con, W_out, b_out):
    emo_rep, con_rep, pred = _projections(
        doc_sents_h, W_emo, b_emo, W_con, b_con, W_out, b_out)
    pred_e = pred.reshape(B, L)

    topk, no_idx, rep_t, ctx_t, rep_n, ctx_n = _indices(pred_e)

    base = (jnp.arange(B, dtype=jnp.int32) * L)[:, None]
    cand, ctx_clause, no_clause, ctx_no_clause = _sc_gather(
        emo_rep.reshape(B * L, D), con_rep.reshape(B * L, D),
        (topk + base).reshape(-1),
        (ctx_t + base).reshape(-1),
        (no_idx + base).reshape(-1),
        (ctx_n + base).reshape(-1),
    )

    pair_t = jnp.stack([rep_t, ctx_t], axis=-1)
    pair_n = jnp.stack([rep_n, ctx_n], axis=-1)
    return (
        pred_e,
        topk,
        pair_t,
        cand.reshape(B, K, D),
        ctx_clause.reshape(B, K, W, D),
        no_clause.reshape(B, LK, D),
        ctx_no_clause.reshape(B, LK, W, D),
        pair_n,
    )


# speedup vs baseline: 4.3406x; 4.3406x over previous
import functools

import jax
import jax.numpy as jnp
from jax import lax
from jax.experimental import pallas as pl
from jax.experimental.pallas import tpu as pltpu
from jax.experimental.pallas import tpu_sc as plsc

B, L, D, K, W = 16, 512, 1024, 32, 5
LK = L - K
BIG = 1 << 30
NEG = -1e30


def _proj_body(doc_ref, we_ref, be_ref, wc_ref, bc_ref, wo_ref, bo_ref,
               emo_ref, con_ref, pred_ref):
    x = doc_ref[0]
    dn = (((1,), (1,)), ((), ()))
    emo = lax.dot_general(x, we_ref[...], dn,
                          preferred_element_type=jnp.float32) + be_ref[...]
    con = lax.dot_general(x, wc_ref[...], dn,
                          preferred_element_type=jnp.float32) + bc_ref[...]
    emo_ref[0] = emo
    con_ref[0] = con
    pred = lax.dot_general(wo_ref[...], emo, dn,
                           preferred_element_type=jnp.float32) + bo_ref[0, 0]
    pred_ref[0] = pred


def _projections(doc, we, be, wc, bc, wo, bo):
    return pl.pallas_call(
        _proj_body,
        grid=(B,),
        in_specs=[
            pl.BlockSpec((1, L, D), lambda b: (b, 0, 0)),
            pl.BlockSpec((D, D), lambda b: (0, 0)),
            pl.BlockSpec((D,), lambda b: (0,)),
            pl.BlockSpec((D, D), lambda b: (0, 0)),
            pl.BlockSpec((D,), lambda b: (0,)),
            pl.BlockSpec((1, D), lambda b: (0, 0)),
            pl.BlockSpec((1, 1), lambda b: (0, 0)),
        ],
        out_specs=[
            pl.BlockSpec((1, L, D), lambda b: (b, 0, 0)),
            pl.BlockSpec((1, L, D), lambda b: (b, 0, 0)),
            pl.BlockSpec((1, 1, L), lambda b: (b, 0, 0)),
        ],
        out_shape=[
            jax.ShapeDtypeStruct((B, L, D), jnp.float32),
            jax.ShapeDtypeStruct((B, L, D), jnp.float32),
            jax.ShapeDtypeStruct((B, 1, L), jnp.float32),
        ],
    )(doc, we, be, wc, bc, wo, bo.reshape(1, 1))


def _index_body(pred_ref, topk_ref, pair_t_ref, pair_n_ref,
                gcand_ref, gctx_t_ref, gno_ref, gctx_n_ref):
    vals = pred_ref[...]
    iota_l = lax.broadcasted_iota(jnp.int32, (B, L), 1)
    iota_k = lax.broadcasted_iota(jnp.int32, (B, K), 1)

    def topk_step(i, carry):
        vals, topk = carry
        m = jnp.max(vals, axis=1, keepdims=True)
        idx = jnp.min(jnp.where(vals == m, iota_l, BIG), axis=1,
                      keepdims=True)
        topk = jnp.where(iota_k == i, idx, topk)
        vals = jnp.where(iota_l == idx, NEG, vals)
        return vals, topk

    topk0 = jnp.zeros((B, K), jnp.int32)
    _, topk = lax.fori_loop(0, K, topk_step, (vals, topk0))
    topk_ref[...] = topk

    def sort_step(i, carry):
        tv, srt = carry
        t = jnp.min(tv, axis=1, keepdims=True)
        srt = jnp.where(iota_k == i, t, srt)
        tv = jnp.where(tv == t, BIG, tv)
        return tv, srt

    _, srt = lax.fori_loop(0, K, sort_step, (topk, topk0))

    v0 = lax.broadcasted_iota(jnp.int32, (B, LK), 1)

    def diff_step(i, v):
        t = jnp.min(jnp.where(iota_k == i, srt, BIG), axis=1, keepdims=True)
        return v + (t <= v).astype(jnp.int32)

    no_idx = lax.fori_loop(0, K, diff_step, v0)

    def expand5(ids, n, w_major):
        src = lax.broadcasted_iota(jnp.int32, (n, W * n), 0)
        j = lax.broadcasted_iota(jnp.int32, (n, W * n), 1)
        dst = (j % n) if w_major else (j // W)
        r = (src == dst).astype(jnp.float32)
        rep = lax.dot_general(ids.astype(jnp.float32), r,
                              (((1,), (0,)), ((), ())),
                              precision=lax.Precision.HIGHEST,
                              preferred_element_type=jnp.float32)
        return rep.astype(jnp.int32)

    def window(rep, n, w_major):
        j = lax.broadcasted_iota(jnp.int32, (B, W * n), 1)
        off = (j // n if w_major else j % W) - 2
        return jnp.clip(rep + off, 0, L - 1)

    def base(n):
        return L * lax.broadcasted_iota(jnp.int32, (B, n), 0)

    rep_t = expand5(topk, K, False)
    rep_n = expand5(no_idx, LK, False)
    pair_t_ref[:, 0, :] = rep_t
    pair_t_ref[:, 1, :] = window(rep_t, K, False)
    pair_n_ref[:, 0, :] = rep_n
    pair_n_ref[:, 1, :] = window(rep_n, LK, False)

    gcand_ref[...] = topk + base(K)
    gno_ref[...] = no_idx + base(LK)
    gctx_t_ref[...] = window(expand5(topk, K, True), K, True) + base(K * W)
    gctx_n_ref[...] = window(expand5(no_idx, LK, True), LK, True) + base(LK * W)


def _indices(pred):
    shapes = [(B, K), (B, 2, K * W), (B, 2, LK * W), (B, K), (B, K * W),
              (B, LK), (B, LK * W)]
    return pl.pallas_call(
        _index_body,
        out_shape=[jax.ShapeDtypeStruct(s, jnp.int32) for s in shapes],
    )(pred)


NW = 32
_CHUNK = 40

_TASKS = (
    ("emo", K * B // NW, 16),
    ("con", K * W * B // NW, 40),
    ("emo", LK * B // NW, 40),
    ("con", LK * W * B // NW, 40),
)


def _sc_gather_body(emo_ref, con_ref, i0_ref, i1_ref, i2_ref, i3_ref,
                    o0_ref, o1_ref, o2_ref, o3_ref,
                    ix0, ix1, ix2, ix3, buf_v, sem_g, sem_o0, sem_o1):
    wid = lax.axis_index("s") * 2 + lax.axis_index("c")
    tables = {"emo": emo_ref, "con": con_ref}
    idx_refs = (i0_ref, i1_ref, i2_ref, i3_ref)
    idx_bufs = (ix0, ix1, ix2, ix3)
    out_refs = (o0_ref, o1_ref, o2_ref, o3_ref)
    sem_o = (sem_o0, sem_o1)

    for t, (_, rows, chunk) in enumerate(_TASKS):
        pltpu.sync_copy(idx_refs[t].at[pl.ds(wid * rows, rows)], idx_bufs[t])

    pending = [None, None]

    def do_chunk(tab, idx_slice, out_slice, slot, chunk):
        if pending[slot] is not None:
            pending[slot].wait()
        pltpu.async_copy(tables[tab].at[idx_slice],
                         buf_v.at[slot, pl.ds(0, chunk)], sem_g).wait()
        pending[slot] = pltpu.async_copy(buf_v.at[slot, pl.ds(0, chunk)],
                                         out_slice, sem_o[slot])

    n = 0
    for t in (0, 1, 2):
        tab, rows, chunk = _TASKS[t]
        for c in range(rows // chunk):
            do_chunk(tab,
                     idx_bufs[t].at[pl.ds(c * chunk, chunk)],
                     out_refs[t].at[pl.ds(wid * rows + c * chunk, chunk)],
                     n % 2, chunk)
            n += 1
    tab, rows, chunk = _TASKS[3]
    nch = rows // chunk

    @pl.loop(0, nch, step=2)
    def _(c0):
        for b in range(2):
            c = c0 + b
            start = pl.multiple_of(c * chunk, chunk)
            obase = pl.multiple_of(wid * rows + c * chunk, chunk)
            pltpu.make_async_copy(buf_v.at[b, pl.ds(0, chunk)],
                                  out_refs[3].at[pl.ds(0, chunk)],
                                  sem_o[b]).wait()
            pltpu.async_copy(tables[tab].at[idx_bufs[3].at[pl.ds(start, chunk)]],
                             buf_v.at[b, pl.ds(0, chunk)], sem_g).wait()
            pltpu.async_copy(buf_v.at[b, pl.ds(0, chunk)],
                             out_refs[3].at[pl.ds(obase, chunk)], sem_o[b])

    for b in range(2):
        pltpu.make_async_copy(buf_v.at[b, pl.ds(0, chunk)],
                              out_refs[3].at[pl.ds(0, chunk)],
                              sem_o[b]).wait()


def _sc_gather(emo_flat, con_flat, g_cand, g_ctx_t, g_no, g_ctx_n):
    mesh = plsc.VectorSubcoreMesh(core_axis_name="c", subcore_axis_name="s")
    kern = pl.kernel(
        _sc_gather_body,
        compiler_params=pltpu.CompilerParams(use_tc_tiling_on_sc=True),
        out_type=[
            jax.ShapeDtypeStruct((B * K, D), jnp.float32),
            jax.ShapeDtypeStruct((B * K * W, D), jnp.float32),
            jax.ShapeDtypeStruct((B * LK, D), jnp.float32),
            jax.ShapeDtypeStruct((B * LK * W, D), jnp.float32),
        ],
        mesh=mesh,
        scratch_types=[
            pltpu.VMEM((K * B // NW,), jnp.int32),
            pltpu.VMEM((K * W * B // NW,), jnp.int32),
            pltpu.VMEM((LK * B // NW,), jnp.int32),
            pltpu.VMEM((LK * W * B // NW,), jnp.int32),
            pltpu.VMEM((2, _CHUNK, D), jnp.float32),
            pltpu.SemaphoreType.DMA,
            pltpu.SemaphoreType.DMA,
            pltpu.SemaphoreType.DMA,
        ],
    )
    return kern(emo_flat, con_flat, g_cand, g_ctx_t, g_no, g_ctx_n)


def kernel(doc_sents_h, W_emo, b_emo, W_con, b_con, W_out, b_out):
    emo_rep, con_rep, pred = _projections(
        doc_sents_h, W_emo, b_emo, W_con, b_con, W_out, b_out)
    pred_e = pred.reshape(B, L)

    topk, pair_t, pair_n, g_cand, g_ctx_t, g_no, g_ctx_n = _indices(pred_e)

    cand, ctx_clause, no_clause, ctx_no_clause = _sc_gather(
        emo_rep.reshape(B * L, D), con_rep.reshape(B * L, D),
        g_cand.reshape(-1), g_ctx_t.reshape(-1),
        g_no.reshape(-1), g_ctx_n.reshape(-1),
    )

    return (
        pred_e,
        topk,
        pair_t.transpose(0, 2, 1),
        cand.reshape(B, K, D),
        ctx_clause.reshape(B, W, K, D).transpose(0, 2, 1, 3),
        no_clause.reshape(B, LK, D),
        ctx_no_clause.reshape(B, W, LK, D).transpose(0, 2, 1, 3),
        pair_n.transpose(0, 2, 1),
    )


# speedup vs baseline: 4.3708x; 1.0070x over previous
import functools

import jax
import jax.numpy as jnp
from jax import lax
from jax.experimental import pallas as pl
from jax.experimental.pallas import tpu as pltpu
from jax.experimental.pallas import tpu_sc as plsc

B, L, D, K, W = 16, 512, 1024, 32, 5
LK = L - K
BIG = 1 << 30
NEG = -1e30


def _proj_index_body(doc_ref, we_ref, be_ref, wc_ref, bc_ref, wo_ref, bo_ref,
                     emo_ref, con_ref, pred_ref, topk_ref, pair_t_ref,
                     pair_n_ref, gcand_ref, gctx_t_ref, gno_ref, gctx_n_ref,
                     pred_v):
    b = pl.program_id(0)
    x = doc_ref[0]
    dn = (((1,), (1,)), ((), ()))
    emo = lax.dot_general(x, we_ref[...], dn,
                          preferred_element_type=jnp.float32) + be_ref[...]
    con = lax.dot_general(x, wc_ref[...], dn,
                          preferred_element_type=jnp.float32) + bc_ref[...]
    emo_ref[0] = emo
    con_ref[0] = con
    pred = lax.dot_general(wo_ref[...], emo, dn,
                           preferred_element_type=jnp.float32) + bo_ref[0, 0]
    pred_v[pl.ds(b, 1), :] = pred

    @pl.when(b == B - 1)
    def _():
        pred_ref[0] = pred_v[...]
        _index_stage(pred_v[...], topk_ref, pair_t_ref, pair_n_ref,
                     gcand_ref, gctx_t_ref, gno_ref, gctx_n_ref)


def _index_stage(vals, topk_ref, pair_t_ref, pair_n_ref,
                 gcand_ref, gctx_t_ref, gno_ref, gctx_n_ref):
    iota_l = lax.broadcasted_iota(jnp.int32, (B, L), 1)
    iota_k = lax.broadcasted_iota(jnp.int32, (B, K), 1)

    def topk_step(i, carry):
        vals, topk = carry
        m = jnp.max(vals, axis=1, keepdims=True)
        idx = jnp.min(jnp.where(vals == m, iota_l, BIG), axis=1,
                      keepdims=True)
        topk = jnp.where(iota_k == i, idx, topk)
        vals = jnp.where(iota_l == idx, NEG, vals)
        return vals, topk

    topk0 = jnp.zeros((B, K), jnp.int32)
    _, topk = lax.fori_loop(0, K, topk_step, (vals, topk0))
    topk_ref[...] = topk

    def sort_step(i, carry):
        tv, srt = carry
        t = jnp.min(tv, axis=1, keepdims=True)
        srt = jnp.where(iota_k == i, t, srt)
        tv = jnp.where(tv == t, BIG, tv)
        return tv, srt

    _, srt = lax.fori_loop(0, K, sort_step, (topk, topk0))

    v0 = lax.broadcasted_iota(jnp.int32, (B, LK), 1)

    def diff_step(i, v):
        t = jnp.min(jnp.where(iota_k == i, srt, BIG), axis=1, keepdims=True)
        return v + (t <= v).astype(jnp.int32)

    no_idx = lax.fori_loop(0, K, diff_step, v0)

    def expand5(ids, n, w_major):
        src = lax.broadcasted_iota(jnp.int32, (n, W * n), 0)
        j = lax.broadcasted_iota(jnp.int32, (n, W * n), 1)
        dst = (j % n) if w_major else (j // W)
        r = (src == dst).astype(jnp.float32)
        rep = lax.dot_general(ids.astype(jnp.float32), r,
                              (((1,), (0,)), ((), ())),
                              precision=lax.Precision.HIGHEST,
                              preferred_element_type=jnp.float32)
        return rep.astype(jnp.int32)

    def window(rep, n, w_major):
        j = lax.broadcasted_iota(jnp.int32, (B, W * n), 1)
        off = (j // n if w_major else j % W) - 2
        return jnp.clip(rep + off, 0, L - 1)

    def base(n):
        return L * lax.broadcasted_iota(jnp.int32, (B, n), 0)

    rep_t = expand5(topk, K, False)
    rep_n = expand5(no_idx, LK, False)
    pair_t_ref[:, 0, :] = rep_t
    pair_t_ref[:, 1, :] = window(rep_t, K, False)
    pair_n_ref[:, 0, :] = rep_n
    pair_n_ref[:, 1, :] = window(rep_n, LK, False)

    gcand_ref[...] = topk + base(K)
    gno_ref[...] = no_idx + base(LK)
    gctx_t_ref[...] = window(expand5(topk, K, True), K, True) + base(K * W)
    gctx_n_ref[...] = window(expand5(no_idx, LK, True), LK, True) + base(LK * W)


def _proj_and_indices(doc, we, be, wc, bc, wo, bo):
    idx_shapes = [(B, K), (B, 2, K * W), (B, 2, LK * W), (B, K), (B, K * W),
                  (B, LK), (B, LK * W)]
    full = lambda s: pl.BlockSpec(s, lambda b: (0,) * len(s))
    return pl.pallas_call(
        _proj_index_body,
        grid=(B,),
        in_specs=[
            pl.BlockSpec((1, L, D), lambda b: (b, 0, 0)),
            full((D, D)),
            full((D,)),
            full((D, D)),
            full((D,)),
            full((1, D)),
            full((1, 1)),
        ],
        out_specs=[
            pl.BlockSpec((1, L, D), lambda b: (b, 0, 0)),
            pl.BlockSpec((1, L, D), lambda b: (b, 0, 0)),
            full((1, B, L)),
        ] + [full(s) for s in idx_shapes],
        out_shape=[
            jax.ShapeDtypeStruct((B, L, D), jnp.float32),
            jax.ShapeDtypeStruct((B, L, D), jnp.float32),
            jax.ShapeDtypeStruct((1, B, L), jnp.float32),
        ] + [jax.ShapeDtypeStruct(s, jnp.int32) for s in idx_shapes],
        scratch_shapes=[pltpu.VMEM((B, L), jnp.float32)],
    )(doc, we, be, wc, bc, wo, bo.reshape(1, 1))


NW = 32
_CHUNK = 40

_TASKS = (
    ("emo", K * B // NW, 16),
    ("con", K * W * B // NW, 40),
    ("emo", LK * B // NW, 40),
    ("con", LK * W * B // NW, 40),
)


def _sc_gather_body(emo_ref, con_ref, i0_ref, i1_ref, i2_ref, i3_ref,
                    o0_ref, o1_ref, o2_ref, o3_ref,
                    ix0, ix1, ix2, ix3, buf_v, sem_g, sem_o0, sem_o1):
    wid = lax.axis_index("s") * 2 + lax.axis_index("c")
    tables = {"emo": emo_ref, "con": con_ref}
    idx_refs = (i0_ref, i1_ref, i2_ref, i3_ref)
    idx_bufs = (ix0, ix1, ix2, ix3)
    out_refs = (o0_ref, o1_ref, o2_ref, o3_ref)
    sem_o = (sem_o0, sem_o1)

    for t, (_, rows, chunk) in enumerate(_TASKS):
        pltpu.sync_copy(idx_refs[t].at[pl.ds(wid * rows, rows)], idx_bufs[t])

    pending = [None, None]

    def do_chunk(tab, idx_slice, out_slice, slot, chunk):
        if pending[slot] is not None:
            pending[slot].wait()
        pltpu.async_copy(tables[tab].at[idx_slice],
                         buf_v.at[slot, pl.ds(0, chunk)], sem_g).wait()
        pending[slot] = pltpu.async_copy(buf_v.at[slot, pl.ds(0, chunk)],
                                         out_slice, sem_o[slot])

    n = 0
    for t in (0, 1, 2):
        tab, rows, chunk = _TASKS[t]
        for c in range(rows // chunk):
            do_chunk(tab,
                     idx_bufs[t].at[pl.ds(c * chunk, chunk)],
                     out_refs[t].at[pl.ds(wid * rows + c * chunk, chunk)],
                     n % 2, chunk)
            n += 1
    tab, rows, chunk = _TASKS[3]
    nch = rows // chunk

    @pl.loop(0, nch, step=2)
    def _(c0):
        for b in range(2):
            c = c0 + b
            start = pl.multiple_of(c * chunk, chunk)
            obase = pl.multiple_of(wid * rows + c * chunk, chunk)
            pltpu.make_async_copy(buf_v.at[b, pl.ds(0, chunk)],
                                  out_refs[3].at[pl.ds(0, chunk)],
                                  sem_o[b]).wait()
            pltpu.async_copy(tables[tab].at[idx_bufs[3].at[pl.ds(start, chunk)]],
                             buf_v.at[b, pl.ds(0, chunk)], sem_g).wait()
            pltpu.async_copy(buf_v.at[b, pl.ds(0, chunk)],
                             out_refs[3].at[pl.ds(obase, chunk)], sem_o[b])

    for b in range(2):
        pltpu.make_async_copy(buf_v.at[b, pl.ds(0, chunk)],
                              out_refs[3].at[pl.ds(0, chunk)],
                              sem_o[b]).wait()


def _sc_gather(emo_flat, con_flat, g_cand, g_ctx_t, g_no, g_ctx_n):
    mesh = plsc.VectorSubcoreMesh(core_axis_name="c", subcore_axis_name="s")
    kern = pl.kernel(
        _sc_gather_body,
        compiler_params=pltpu.CompilerParams(use_tc_tiling_on_sc=True),
        out_type=[
            jax.ShapeDtypeStruct((B * K, D), jnp.float32),
            jax.ShapeDtypeStruct((B * K * W, D), jnp.float32),
            jax.ShapeDtypeStruct((B * LK, D), jnp.float32),
            jax.ShapeDtypeStruct((B * LK * W, D), jnp.float32),
        ],
        mesh=mesh,
        scratch_types=[
            pltpu.VMEM((K * B // NW,), jnp.int32),
            pltpu.VMEM((K * W * B // NW,), jnp.int32),
            pltpu.VMEM((LK * B // NW,), jnp.int32),
            pltpu.VMEM((LK * W * B // NW,), jnp.int32),
            pltpu.VMEM((2, _CHUNK, D), jnp.float32),
            pltpu.SemaphoreType.DMA,
            pltpu.SemaphoreType.DMA,
            pltpu.SemaphoreType.DMA,
        ],
    )
    return kern(emo_flat, con_flat, g_cand, g_ctx_t, g_no, g_ctx_n)


def kernel(doc_sents_h, W_emo, b_emo, W_con, b_con, W_out, b_out):
    (emo_rep, con_rep, pred, topk, pair_t, pair_n, g_cand, g_ctx_t, g_no,
     g_ctx_n) = _proj_and_indices(
        doc_sents_h, W_emo, b_emo, W_con, b_con, W_out, b_out)
    pred_e = pred.reshape(B, L)

    cand, ctx_clause, no_clause, ctx_no_clause = _sc_gather(
        emo_rep.reshape(B * L, D), con_rep.reshape(B * L, D),
        g_cand.reshape(-1), g_ctx_t.reshape(-1),
        g_no.reshape(-1), g_ctx_n.reshape(-1),
    )

    return (
        pred_e,
        topk,
        pair_t.transpose(0, 2, 1),
        cand.reshape(B, K, D),
        ctx_clause.reshape(B, W, K, D).transpose(0, 2, 1, 3),
        no_clause.reshape(B, LK, D),
        ctx_no_clause.reshape(B, W, LK, D).transpose(0, 2, 1, 3),
        pair_n.transpose(0, 2, 1),
    )


# speedup vs baseline: 4.4034x; 1.0075x over previous
import functools

import jax
import jax.numpy as jnp
from jax import lax
from jax.experimental import pallas as pl
from jax.experimental.pallas import tpu as pltpu
from jax.experimental.pallas import tpu_sc as plsc

B, L, D, K, W = 16, 512, 1024, 32, 5
LK = L - K
BIG = 1 << 30
NEG = -1e30


def _proj_index_body(doc_ref, we_ref, be_ref, wc_ref, bc_ref, wo_ref, bo_ref,
                     emo_ref, con_ref, pred_ref, topk_ref, pair_t_ref,
                     pair_n_ref, gcand_ref, gctx_t_ref, gno_ref, gctx_n_ref,
                     pred_v):
    b = pl.program_id(0)
    x = doc_ref[0]
    dn = (((1,), (1,)), ((), ()))
    emo = lax.dot_general(x, we_ref[...], dn,
                          preferred_element_type=jnp.float32) + be_ref[...]
    con = lax.dot_general(x, wc_ref[...], dn,
                          preferred_element_type=jnp.float32) + bc_ref[...]
    emo_ref[0] = emo
    con_ref[0] = con
    pred = lax.dot_general(wo_ref[...], emo, dn,
                           preferred_element_type=jnp.float32) + bo_ref[0, 0]
    pred_v[pl.ds(b, 1), :] = pred

    @pl.when(b == B - 1)
    def _():
        pred_ref[0] = pred_v[...]
        _index_stage(pred_v[...], topk_ref, pair_t_ref, pair_n_ref,
                     gcand_ref, gctx_t_ref, gno_ref, gctx_n_ref)


def _index_stage(vals, topk_ref, pair_t_ref, pair_n_ref,
                 gcand_ref, gctx_t_ref, gno_ref, gctx_n_ref):
    iota_l = lax.broadcasted_iota(jnp.int32, (B, L), 1)
    iota_k = lax.broadcasted_iota(jnp.int32, (B, K), 1)

    def topk_step(i, carry):
        vals, topk = carry
        m = jnp.max(vals, axis=1, keepdims=True)
        idx = jnp.min(jnp.where(vals == m, iota_l, BIG), axis=1,
                      keepdims=True)
        topk = jnp.where(iota_k == i, idx, topk)
        vals = jnp.where(iota_l == idx, NEG, vals)
        return vals, topk

    topk0 = jnp.zeros((B, K), jnp.int32)
    _, topk = lax.fori_loop(0, K, topk_step, (vals, topk0))
    topk_ref[...] = topk

    def sort_step(i, carry):
        tv, srt = carry
        t = jnp.min(tv, axis=1, keepdims=True)
        srt = jnp.where(iota_k == i, t, srt)
        tv = jnp.where(tv == t, BIG, tv)
        return tv, srt

    _, srt = lax.fori_loop(0, K, sort_step, (topk, topk0))

    v0 = lax.broadcasted_iota(jnp.int32, (B, LK), 1)

    def diff_step(i, v):
        t = jnp.min(jnp.where(iota_k == i, srt, BIG), axis=1, keepdims=True)
        return v + (t <= v).astype(jnp.int32)

    no_idx = lax.fori_loop(0, K, diff_step, v0)

    def expand5(ids, n, w_major):
        src = lax.broadcasted_iota(jnp.int32, (n, W * n), 0)
        j = lax.broadcasted_iota(jnp.int32, (n, W * n), 1)
        dst = (j % n) if w_major else (j // W)
        r = (src == dst).astype(jnp.float32)
        rep = lax.dot_general(ids.astype(jnp.float32), r,
                              (((1,), (0,)), ((), ())),
                              precision=lax.Precision.HIGHEST,
                              preferred_element_type=jnp.float32)
        return rep.astype(jnp.int32)

    def window(rep, n, w_major):
        j = lax.broadcasted_iota(jnp.int32, (B, W * n), 1)
        off = (j // n if w_major else j % W) - 2
        return jnp.clip(rep + off, 0, L - 1)

    def base(n):
        return L * lax.broadcasted_iota(jnp.int32, (B, n), 0)

    rep_t = expand5(topk, K, False)
    rep_n = expand5(no_idx, LK, False)
    pair_t_ref[:, 0, :] = rep_t
    pair_t_ref[:, 1, :] = window(rep_t, K, False)
    pair_n_ref[:, 0, :] = rep_n
    pair_n_ref[:, 1, :] = window(rep_n, LK, False)

    def ctx_w(ids, n):
        return jnp.concatenate(
            [jnp.clip(ids + o, 0, L - 1) for o in range(-2, 3)], axis=1)

    gcand_ref[...] = topk + base(K)
    gno_ref[...] = no_idx + base(LK)
    gctx_t_ref[...] = ctx_w(topk, K) + base(K * W)
    gctx_n_ref[...] = ctx_w(no_idx, LK) + base(LK * W)


def _proj_and_indices(doc, we, be, wc, bc, wo, bo):
    idx_shapes = [(B, K), (B, 2, K * W), (B, 2, LK * W), (B, K), (B, K * W),
                  (B, LK), (B, LK * W)]
    full = lambda s: pl.BlockSpec(s, lambda b: (0,) * len(s))
    return pl.pallas_call(
        _proj_index_body,
        grid=(B,),
        in_specs=[
            pl.BlockSpec((1, L, D), lambda b: (b, 0, 0)),
            full((D, D)),
            full((D,)),
            full((D, D)),
            full((D,)),
            full((1, D)),
            full((1, 1)),
        ],
        out_specs=[
            pl.BlockSpec((1, L, D), lambda b: (b, 0, 0)),
            pl.BlockSpec((1, L, D), lambda b: (b, 0, 0)),
            full((1, B, L)),
        ] + [full(s) for s in idx_shapes],
        out_shape=[
            jax.ShapeDtypeStruct((B, L, D), jnp.float32),
            jax.ShapeDtypeStruct((B, L, D), jnp.float32),
            jax.ShapeDtypeStruct((1, B, L), jnp.float32),
        ] + [jax.ShapeDtypeStruct(s, jnp.int32) for s in idx_shapes],
        scratch_shapes=[pltpu.VMEM((B, L), jnp.float32)],
    )(doc, we, be, wc, bc, wo, bo.reshape(1, 1))


NW = 32
_CHUNK = 40

_TASKS = (
    ("emo", K * B // NW, 16),
    ("con", K * W * B // NW, 40),
    ("emo", LK * B // NW, 40),
    ("con", LK * W * B // NW, 40),
)


def _sc_gather_body(emo_ref, con_ref, i0_ref, i1_ref, i2_ref, i3_ref,
                    o0_ref, o1_ref, o2_ref, o3_ref,
                    ix0, ix1, ix2, ix3, buf_v, sem_g, sem_o0, sem_o1):
    wid = lax.axis_index("s") * 2 + lax.axis_index("c")
    tables = {"emo": emo_ref, "con": con_ref}
    idx_refs = (i0_ref, i1_ref, i2_ref, i3_ref)
    idx_bufs = (ix0, ix1, ix2, ix3)
    out_refs = (o0_ref, o1_ref, o2_ref, o3_ref)
    sem_o = (sem_o0, sem_o1)

    for t, (_, rows, chunk) in enumerate(_TASKS):
        pltpu.sync_copy(idx_refs[t].at[pl.ds(wid * rows, rows)], idx_bufs[t])

    pending = [None, None]

    def do_chunk(tab, idx_slice, out_slice, slot, chunk):
        if pending[slot] is not None:
            pending[slot].wait()
        pltpu.async_copy(tables[tab].at[idx_slice],
                         buf_v.at[slot, pl.ds(0, chunk)], sem_g).wait()
        pending[slot] = pltpu.async_copy(buf_v.at[slot, pl.ds(0, chunk)],
                                         out_slice, sem_o[slot])

    n = 0
    for t in (0, 1, 2):
        tab, rows, chunk = _TASKS[t]
        for c in range(rows // chunk):
            do_chunk(tab,
                     idx_bufs[t].at[pl.ds(c * chunk, chunk)],
                     out_refs[t].at[pl.ds(wid * rows + c * chunk, chunk)],
                     n % 2, chunk)
            n += 1
    tab, rows, chunk = _TASKS[3]
    nch = rows // chunk

    @pl.loop(0, nch, step=2)
    def _(c0):
        for b in range(2):
            c = c0 + b
            start = pl.multiple_of(c * chunk, chunk)
            obase = pl.multiple_of(wid * rows + c * chunk, chunk)
            pltpu.make_async_copy(buf_v.at[b, pl.ds(0, chunk)],
                                  out_refs[3].at[pl.ds(0, chunk)],
                                  sem_o[b]).wait()
            pltpu.async_copy(tables[tab].at[idx_bufs[3].at[pl.ds(start, chunk)]],
                             buf_v.at[b, pl.ds(0, chunk)], sem_g).wait()
            pltpu.async_copy(buf_v.at[b, pl.ds(0, chunk)],
                             out_refs[3].at[pl.ds(obase, chunk)], sem_o[b])

    for b in range(2):
        pltpu.make_async_copy(buf_v.at[b, pl.ds(0, chunk)],
                              out_refs[3].at[pl.ds(0, chunk)],
                              sem_o[b]).wait()


def _sc_gather(emo_flat, con_flat, g_cand, g_ctx_t, g_no, g_ctx_n):
    mesh = plsc.VectorSubcoreMesh(core_axis_name="c", subcore_axis_name="s")
    kern = pl.kernel(
        _sc_gather_body,
        compiler_params=pltpu.CompilerParams(use_tc_tiling_on_sc=True),
        out_type=[
            jax.ShapeDtypeStruct((B * K, D), jnp.float32),
            jax.ShapeDtypeStruct((B * K * W, D), jnp.float32),
            jax.ShapeDtypeStruct((B * LK, D), jnp.float32),
            jax.ShapeDtypeStruct((B * LK * W, D), jnp.float32),
        ],
        mesh=mesh,
        scratch_types=[
            pltpu.VMEM((K * B // NW,), jnp.int32),
            pltpu.VMEM((K * W * B // NW,), jnp.int32),
            pltpu.VMEM((LK * B // NW,), jnp.int32),
            pltpu.VMEM((LK * W * B // NW,), jnp.int32),
            pltpu.VMEM((2, _CHUNK, D), jnp.float32),
            pltpu.SemaphoreType.DMA,
            pltpu.SemaphoreType.DMA,
            pltpu.SemaphoreType.DMA,
        ],
    )
    return kern(emo_flat, con_flat, g_cand, g_ctx_t, g_no, g_ctx_n)


def kernel(doc_sents_h, W_emo, b_emo, W_con, b_con, W_out, b_out):
    (emo_rep, con_rep, pred, topk, pair_t, pair_n, g_cand, g_ctx_t, g_no,
     g_ctx_n) = _proj_and_indices(
        doc_sents_h, W_emo, b_emo, W_con, b_con, W_out, b_out)
    pred_e = pred.reshape(B, L)

    cand, ctx_clause, no_clause, ctx_no_clause = _sc_gather(
        emo_rep.reshape(B * L, D), con_rep.reshape(B * L, D),
        g_cand.reshape(-1), g_ctx_t.reshape(-1),
        g_no.reshape(-1), g_ctx_n.reshape(-1),
    )

    return (
        pred_e,
        topk,
        pair_t.transpose(0, 2, 1),
        cand.reshape(B, K, D),
        ctx_clause.reshape(B, W, K, D).transpose(0, 2, 1, 3),
        no_clause.reshape(B, LK, D),
        ctx_no_clause.reshape(B, W, LK, D).transpose(0, 2, 1, 3),
        pair_n.transpose(0, 2, 1),
    )


# speedup vs baseline: 4.5722x; 1.0383x over previous
import functools

import jax
import jax.numpy as jnp
from jax import lax
from jax.experimental import pallas as pl
from jax.experimental.pallas import tpu as pltpu
from jax.experimental.pallas import tpu_sc as plsc

B, L, D, K, W = 16, 512, 1024, 32, 5
LK = L - K
BIG = 1 << 30
NEG = -1e30


def _proj_index_body(doc_ref, we_ref, be_ref, wc_ref, bc_ref, wo_ref, bo_ref,
                     emo_ref, con_ref, pred_ref, topk_ref, pair_t_ref,
                     pair_n_ref, gcand_ref, gctx_t_ref, gno_ref, gctx_n_ref,
                     pred_v):
    b = pl.program_id(0)
    x = doc_ref[0]
    dn = (((1,), (1,)), ((), ()))
    emo = lax.dot_general(x, we_ref[...], dn,
                          preferred_element_type=jnp.float32) + be_ref[...]
    con = lax.dot_general(x, wc_ref[...], dn,
                          preferred_element_type=jnp.float32) + bc_ref[...]
    emo_ref[0] = emo
    con_ref[0] = con
    pred = lax.dot_general(wo_ref[...], emo, dn,
                           preferred_element_type=jnp.float32) + bo_ref[0, 0]
    pred_v[pl.ds(b, 1), :] = pred

    @pl.when(b == B - 1)
    def _():
        pred_ref[0] = pred_v[...]
        _index_stage(pred_v[...], topk_ref, pair_t_ref, pair_n_ref,
                     gcand_ref, gctx_t_ref, gno_ref, gctx_n_ref)


def _index_stage(vals, topk_ref, pair_t_ref, pair_n_ref,
                 gcand_ref, gctx_t_ref, gno_ref, gctx_n_ref):
    iota_l = lax.broadcasted_iota(jnp.int32, (B, L), 1)
    iota_k = lax.broadcasted_iota(jnp.int32, (B, K), 1)

    def topk_step(i, carry):
        vals, topk = carry
        m = jnp.max(vals, axis=1, keepdims=True)
        idx = jnp.min(jnp.where(vals == m, iota_l, BIG), axis=1,
                      keepdims=True)
        topk = jnp.where(iota_k == i, idx, topk)
        vals = jnp.where(iota_l == idx, NEG, vals)
        return vals, topk

    topk0 = jnp.zeros((B, K), jnp.int32)
    _, topk = lax.fori_loop(0, K, topk_step, (vals, topk0))
    topk_ref[...] = topk

    def sort_step(i, carry):
        tv, srt = carry
        t = jnp.min(tv, axis=1, keepdims=True)
        srt = jnp.where(iota_k == i, t, srt)
        tv = jnp.where(tv == t, BIG, tv)
        return tv, srt

    _, srt = lax.fori_loop(0, K, sort_step, (topk, topk0))

    v0 = lax.broadcasted_iota(jnp.int32, (B, LK), 1)

    def diff_step(i, v):
        t = jnp.min(jnp.where(iota_k == i, srt, BIG), axis=1, keepdims=True)
        return v + (t <= v).astype(jnp.int32)

    no_idx = lax.fori_loop(0, K, diff_step, v0)

    def expand5(ids, n, w_major):
        src = lax.broadcasted_iota(jnp.int32, (n, W * n), 0)
        j = lax.broadcasted_iota(jnp.int32, (n, W * n), 1)
        dst = (j % n) if w_major else (j // W)
        r = (src == dst).astype(jnp.float32)
        rep = lax.dot_general(ids.astype(jnp.float32), r,
                              (((1,), (0,)), ((), ())),
                              precision=lax.Precision.HIGHEST,
                              preferred_element_type=jnp.float32)
        return rep.astype(jnp.int32)

    def window(rep, n, w_major):
        j = lax.broadcasted_iota(jnp.int32, (B, W * n), 1)
        off = (j // n if w_major else j % W) - 2
        return jnp.clip(rep + off, 0, L - 1)

    def base(n):
        return L * lax.broadcasted_iota(jnp.int32, (B, n), 0)

    rep_t = expand5(topk, K, False)
    rep_n = expand5(no_idx, LK, False)
    pair_t_ref[:, 0, :] = rep_t
    pair_t_ref[:, 1, :] = window(rep_t, K, False)
    pair_n_ref[:, 0, :] = rep_n
    pair_n_ref[:, 1, :] = window(rep_n, LK, False)

    def ctx_w(ids, n):
        return jnp.concatenate(
            [jnp.clip(ids + o, 0, L - 1) for o in range(-2, 3)], axis=1)

    gcand_ref[...] = topk + base(K)
    gno_ref[...] = no_idx + base(LK)
    gctx_t_ref[...] = ctx_w(topk, K) + base(K * W)
    gctx_n_ref[...] = ctx_w(no_idx, LK) + base(LK * W)


def _proj_and_indices(doc, we, be, wc, bc, wo, bo):
    idx_shapes = [(B, K), (B, 2, K * W), (B, 2, LK * W), (B, K), (B, K * W),
                  (B, LK), (B, LK * W)]
    full = lambda s: pl.BlockSpec(s, lambda b: (0,) * len(s))
    return pl.pallas_call(
        _proj_index_body,
        grid=(B,),
        in_specs=[
            pl.BlockSpec((1, L, D), lambda b: (b, 0, 0)),
            full((D, D)),
            full((D,)),
            full((D, D)),
            full((D,)),
            full((1, D)),
            full((1, 1)),
        ],
        out_specs=[
            pl.BlockSpec((1, L, D), lambda b: (b, 0, 0)),
            pl.BlockSpec((1, L, D), lambda b: (b, 0, 0)),
            full((1, B, L)),
        ] + [full(s) for s in idx_shapes],
        out_shape=[
            jax.ShapeDtypeStruct((B, L, D), jnp.float32),
            jax.ShapeDtypeStruct((B, L, D), jnp.float32),
            jax.ShapeDtypeStruct((1, B, L), jnp.float32),
        ] + [jax.ShapeDtypeStruct(s, jnp.int32) for s in idx_shapes],
        scratch_shapes=[pltpu.VMEM((B, L), jnp.float32)],
    )(doc, we, be, wc, bc, wo, bo.reshape(1, 1))


NW = 32
_CHUNK = 40

_TASKS = (
    (K * W * B // NW, 40),
    (LK * W * B // NW, 40),
)


def _sc_gather_body(con_ref, i0_ref, i1_ref, o0_ref, o1_ref,
                    ix0, ix1, buf_v, sem_g, sem_o0, sem_o1):
    wid = lax.axis_index("s") * 2 + lax.axis_index("c")
    idx_refs = (i0_ref, i1_ref)
    idx_bufs = (ix0, ix1)
    out_refs = (o0_ref, o1_ref)
    sem_o = (sem_o0, sem_o1)

    for t, (rows, chunk) in enumerate(_TASKS):
        pltpu.sync_copy(idx_refs[t].at[pl.ds(wid * rows, rows)], idx_bufs[t])

    pending = [None, None]

    def do_chunk(idx_slice, out_slice, slot, chunk):
        if pending[slot] is not None:
            pending[slot].wait()
        pltpu.async_copy(con_ref.at[idx_slice],
                         buf_v.at[slot, pl.ds(0, chunk)], sem_g).wait()
        pending[slot] = pltpu.async_copy(buf_v.at[slot, pl.ds(0, chunk)],
                                         out_slice, sem_o[slot])

    n = 0
    rows, chunk = _TASKS[0]
    for c in range(rows // chunk):
        do_chunk(idx_bufs[0].at[pl.ds(c * chunk, chunk)],
                 out_refs[0].at[pl.ds(wid * rows + c * chunk, chunk)],
                 n % 2, chunk)
        n += 1
    rows, chunk = _TASKS[1]
    nch = rows // chunk

    @pl.loop(0, nch, step=2)
    def _(c0):
        for b in range(2):
            c = c0 + b
            start = pl.multiple_of(c * chunk, chunk)
            obase = pl.multiple_of(wid * rows + c * chunk, chunk)
            pltpu.make_async_copy(buf_v.at[b, pl.ds(0, chunk)],
                                  out_refs[1].at[pl.ds(0, chunk)],
                                  sem_o[b]).wait()
            pltpu.async_copy(con_ref.at[idx_bufs[1].at[pl.ds(start, chunk)]],
                             buf_v.at[b, pl.ds(0, chunk)], sem_g).wait()
            pltpu.async_copy(buf_v.at[b, pl.ds(0, chunk)],
                             out_refs[1].at[pl.ds(obase, chunk)], sem_o[b])

    for b in range(2):
        pltpu.make_async_copy(buf_v.at[b, pl.ds(0, chunk)],
                              out_refs[1].at[pl.ds(0, chunk)],
                              sem_o[b]).wait()


def _sc_gather(con_flat, g_ctx_t, g_ctx_n):
    mesh = plsc.VectorSubcoreMesh(core_axis_name="c", subcore_axis_name="s")
    kern = pl.kernel(
        _sc_gather_body,
        compiler_params=pltpu.CompilerParams(use_tc_tiling_on_sc=True),
        out_type=[
            jax.ShapeDtypeStruct((B * K * W, D), jnp.float32),
            jax.ShapeDtypeStruct((B * LK * W, D), jnp.float32),
        ],
        mesh=mesh,
        scratch_types=[
            pltpu.VMEM((K * W * B // NW,), jnp.int32),
            pltpu.VMEM((LK * W * B // NW,), jnp.int32),
            pltpu.VMEM((2, _CHUNK, D), jnp.float32),
            pltpu.SemaphoreType.DMA,
            pltpu.SemaphoreType.DMA,
            pltpu.SemaphoreType.DMA,
        ],
    )
    return kern(con_flat, g_ctx_t, g_ctx_n)


def _tc_gather_body(gcand_ref, gno_ref, emo_ref, cand_ref, no_ref):
    base = pl.program_id(0) * L

    def run(idx_ref, out_ref, n):
        def cp(i, carry):
            r = idx_ref[0, 0, i] - base
            out_ref[0, pl.ds(i, 1), :] = emo_ref[0, pl.ds(r, 1), :]
            return carry

        lax.fori_loop(0, n, cp, 0)

    run(gcand_ref, cand_ref, K)
    run(gno_ref, no_ref, LK)


def _tc_gather(g_cand, g_no, emo_rep):
    return pl.pallas_call(
        _tc_gather_body,
        grid=(B,),
        in_specs=[
            pl.BlockSpec((1, 1, K), lambda b: (b, 0, 0),
                         memory_space=pltpu.SMEM),
            pl.BlockSpec((1, 1, LK), lambda b: (b, 0, 0),
                         memory_space=pltpu.SMEM),
            pl.BlockSpec((1, L, D), lambda b: (b, 0, 0)),
        ],
        out_specs=[
            pl.BlockSpec((1, K, D), lambda b: (b, 0, 0)),
            pl.BlockSpec((1, LK, D), lambda b: (b, 0, 0)),
        ],
        out_shape=[
            jax.ShapeDtypeStruct((B, K, D), jnp.float32),
            jax.ShapeDtypeStruct((B, LK, D), jnp.float32),
        ],
    )(g_cand.reshape(B, 1, K), g_no.reshape(B, 1, LK), emo_rep)


def kernel(doc_sents_h, W_emo, b_emo, W_con, b_con, W_out, b_out):
    (emo_rep, con_rep, pred, topk, pair_t, pair_n, g_cand, g_ctx_t, g_no,
     g_ctx_n) = _proj_and_indices(
        doc_sents_h, W_emo, b_emo, W_con, b_con, W_out, b_out)
    pred_e = pred.reshape(B, L)

    ctx_clause, ctx_no_clause = _sc_gather(
        con_rep.reshape(B * L, D),
        g_ctx_t.reshape(-1), g_ctx_n.reshape(-1),
    )
    cand, no_clause = _tc_gather(g_cand, g_no, emo_rep)

    return (
        pred_e,
        topk,
        pair_t.transpose(0, 2, 1),
        cand,
        ctx_clause.reshape(B, W, K, D).transpose(0, 2, 1, 3),
        no_clause,
        ctx_no_clause.reshape(B, W, LK, D).transpose(0, 2, 1, 3),
        pair_n.transpose(0, 2, 1),
    )
